# baseline (speedup 1.0000x reference)
import functools

import jax
import jax.numpy as jnp
from jax import lax
from jax.experimental import pallas as pl
from jax.experimental.pallas import tpu as pltpu
from jax.experimental.pallas import tpu_sc as plsc

N = 10000
E = 160000
F = 256
H = 256
C = 16

BM = 256
BX = 400
NP = 10240
NB = NP // BM
EP = 163840
EB = EP // 128
CH = 5
Q = 64
N_SUBCORES = 16
ROWS_PER_TILE = EB // N_SUBCORES
CHUNKS_PER_TILE = ROWS_PER_TILE // CH
STRIPE = NP // N_SUBCORES


def _segsum_body(table, src_f, dst_f, src_r, dst_r, zblk,
                 agg_f, agg_r, idx_s, idx_d, rows, acc,
                 sem_g0, sem_g1, sem_s0, sem_s1):
    c = lax.axis_index("c")
    s = lax.axis_index("s")
    sem_g = (sem_g0, sem_g1)
    sem_s = (sem_s0, sem_s1)

    def run_phase(src_ref, dst_ref, out_ref, q):
        base = s * ROWS_PER_TILE
        for z in range(STRIPE // 128):
            pltpu.sync_copy(zblk, acc.at[pl.ds(s * STRIPE + z * 128, 128)])
        plsc.subcore_barrier()

        def fire_gather(k, b):
            r0 = base + k * CH
            pltpu.sync_copy(src_ref.at[q, pl.ds(r0, CH)], idx_s.at[b])
            pltpu.sync_copy(dst_ref.at[pl.ds(r0, CH)], idx_d.at[b])
            for j in range(CH):
                pltpu.async_copy(table.at[idx_s.at[b, j]], rows.at[b, j],
                                 sem_g[b])

        def drain_gather(b):
            for j in range(CH):
                pltpu.make_async_copy(table.at[idx_s.at[b, j]],
                                      rows.at[b, j], sem_g[b]).wait()

        def fire_scatter(b):
            for j in range(CH):
                pltpu.async_copy(rows.at[b, j], acc.at[idx_d.at[b, j]],
                                 sem_s[b], add=True)

        def drain_scatter(b):
            for j in range(CH):
                pltpu.make_async_copy(rows.at[b, j], acc.at[idx_d.at[b, j]],
                                      sem_s[b]).wait()

        fire_gather(0, 0)

        def outer(i, carry):
            for b in range(2):
                k = 2 * i + b

                @pl.when(k >= 1)
                def _():
                    drain_scatter(1 - b)

                @pl.when(k + 1 < CHUNKS_PER_TILE)
                def _():
                    fire_gather(k + 1, 1 - b)

                drain_gather(b)
                fire_scatter(b)
            return carry

        lax.fori_loop(0, CHUNKS_PER_TILE // 2, outer, 0)
        drain_scatter(1)
        plsc.subcore_barrier()
        pltpu.sync_copy(acc.at[pl.ds(s * STRIPE, STRIPE)],
                        out_ref.at[q, pl.ds(s * STRIPE, STRIPE)])

    for p in range(2):
        run_phase(src_f, dst_f, agg_f, c * 2 + p)
        run_phase(src_r, dst_r, agg_r, c * 2 + p)


def _make_segsum():
    mesh = plsc.VectorSubcoreMesh(core_axis_name="c", subcore_axis_name="s")
    return pl.kernel(
        _segsum_body,
        out_type=(
            jax.ShapeDtypeStruct((4, NP, Q), jnp.float32),
            jax.ShapeDtypeStruct((4, NP, Q), jnp.float32),
        ),
        mesh=mesh,
        scratch_types=[
            pltpu.VMEM((2, CH, 128), jnp.int32),
            pltpu.VMEM((2, CH, 128), jnp.int32),
            pltpu.VMEM((2, CH, 128, Q), jnp.float32),
            pltpu.VMEM_SHARED((NP, Q), jnp.float32),
            pltpu.SemaphoreType.DMA,
            pltpu.SemaphoreType.DMA,
            pltpu.SemaphoreType.DMA,
            pltpu.SemaphoreType.DMA,
        ],
        compiler_params=pltpu.CompilerParams(use_tc_tiling_on_sc=False),
    )


def _split4(o_ref, h):
    for q in range(4):
        o_ref[q] = h[:, q * Q:(q + 1) * Q]


def _cat(r4):
    return jnp.concatenate([r4[0], r4[1], r4[2], r4[3]], axis=1)


_dot = functools.partial(jnp.dot, preferred_element_type=jnp.float32)


def _first_body(x_ref, wf_ref, bf_ref, wa_ref, wb_ref, ba_ref, bb_ref,
                h_ref, ra_ref, rb_ref):
    h = jnp.maximum(_dot(x_ref[...], wf_ref[...]) + bf_ref[...], 0.0)
    _split4(h_ref, h)
    ra_ref[...] = _dot(h, wa_ref[...]) + ba_ref[...]
    rb_ref[...] = _dot(h, wb_ref[...]) + bb_ref[...]


def _first_tc(x, WfT, bf, WtaT, WtbT, ba, bb):
    full = lambda a, b: pl.BlockSpec((a, b), lambda i: (0, 0))
    nbx = N // BX
    return pl.pallas_call(
        _first_body,
        grid=(nbx,),
        in_specs=[
            pl.BlockSpec((BX, F), lambda i: (i, 0)),
            full(F, H), full(1, H),
            full(H, H), full(H, H), full(1, H), full(1, H),
        ],
        out_specs=(
            pl.BlockSpec((4, BX, Q), lambda i: (0, i, 0)),
            pl.BlockSpec((BX, H), lambda i: (i, 0)),
            pl.BlockSpec((BX, H), lambda i: (i, 0)),
        ),
        out_shape=(
            jax.ShapeDtypeStruct((4, NP, Q), jnp.float32),
            jax.ShapeDtypeStruct((NP, H), jnp.float32),
            jax.ShapeDtypeStruct((NP, H), jnp.float32),
        ),
    )(x, WfT, bf, WtaT, WtbT, ba, bb)


def _layer1_body(af_ref, ar_ref, rf_ref, rr_ref,
                 wr1_ref, wr1d_ref, wcon_ref, bcon_ref,
                 wa_ref, wb_ref, ba_ref, bb_ref,
                 h1_ref, ra_ref, rb_ref):
    af = _cat(af_ref[...])
    ar = _cat(ar_ref[...])
    x1 = jnp.maximum(_dot(af, wr1_ref[...]) + rf_ref[...], 0.0)
    x2 = jnp.maximum(_dot(ar, wr1d_ref[...]) + rr_ref[...], 0.0)
    wcon = wcon_ref[...]
    h1 = _dot(x1, wcon[:H]) + _dot(x2, wcon[H:]) + bcon_ref[...]
    _split4(h1_ref, h1)
    ra_ref[...] = _dot(h1, wa_ref[...]) + ba_ref[...]
    rb_ref[...] = _dot(h1, wb_ref[...]) + bb_ref[...]


def _layer1_tc(aggf, aggr, rootf, rootr, Wr1T, Wr1dT, WconT, bcon,
               Wt2T, Wt2dT, b2, b2d):
    full = lambda a, b: pl.BlockSpec((a, b), lambda i: (0, 0))
    blk4 = pl.BlockSpec((4, BM, Q), lambda i: (0, i, 0))
    blk2 = pl.BlockSpec((BM, H), lambda i: (i, 0))
    return pl.pallas_call(
        _layer1_body,
        grid=(NB,),
        in_specs=[blk4, blk4, blk2, blk2,
                  full(H, H), full(H, H), full(2 * H, H), full(1, H),
                  full(H, H), full(H, H), full(1, H), full(1, H)],
        out_specs=(
            pl.BlockSpec((4, BM, Q), lambda i: (0, i, 0)),
            blk2, blk2,
        ),
        out_shape=(
            jax.ShapeDtypeStruct((4, NP, Q), jnp.float32),
            jax.ShapeDtypeStruct((NP, H), jnp.float32),
            jax.ShapeDtypeStruct((NP, H), jnp.float32),
        ),
    )(aggf, aggr, rootf, rootr, Wr1T, Wr1dT, WconT, bcon,
      Wt2T, Wt2dT, b2, b2d)


def _layer2_body(af_ref, ar_ref, rf_ref, rr_ref,
                 wr2_ref, wr2d_ref, wout_ref, bout_ref, o_ref):
    af = _cat(af_ref[...])
    ar = _cat(ar_ref[...])
    x1 = jnp.maximum(_dot(af, wr2_ref[...]) + rf_ref[...], 0.0)
    x2 = jnp.maximum(_dot(ar, wr2d_ref[...]) + rr_ref[...], 0.0)
    wout = wout_ref[...]
    logits = _dot(x1, wout[:H]) + _dot(x2, wout[H:]) + bout_ref[...]
    m = jnp.max(logits, axis=-1, keepdims=True)
    z = logits - m
    lse = jnp.log(jnp.sum(jnp.exp(z), axis=-1, keepdims=True))
    o_ref[...] = z - lse


def _layer2_tc(aggf, aggr, rootf, rootr, Wr2T, Wr2dT, WoutT, bout):
    full = lambda a, b: pl.BlockSpec((a, b), lambda i: (0, 0))
    blk4 = pl.BlockSpec((4, BM, Q), lambda i: (0, i, 0))
    blk2 = pl.BlockSpec((BM, H), lambda i: (i, 0))
    return pl.pallas_call(
        _layer2_body,
        grid=(NB,),
        in_specs=[blk4, blk4, blk2, blk2,
                  full(H, H), full(H, H), full(2 * H, C), full(1, C)],
        out_specs=pl.BlockSpec((BM, C), lambda i: (i, 0)),
        out_shape=jax.ShapeDtypeStruct((NP, C), jnp.float32),
    )(aggf, aggr, rootf, rootr, Wr2T, Wr2dT, WoutT, bout)


def _prep_edges(ei):
    src = jnp.full((EP,), N, dtype=jnp.int32).at[:E].set(ei[0])
    dst = jnp.full((EP,), N, dtype=jnp.int32).at[:E].set(ei[1])
    offs = jnp.arange(4, dtype=jnp.int32)[:, None] * NP
    srcq = (src[None, :] + offs).reshape(4, EB, 128)
    return srcq, dst.reshape(EB, 128)


def kernel(x, edge_index, edge_weight, edge_index_re, edge_weight_re,
           W_first, b_first,
           W_rel1, b_rel1, W_root1,
           W_rel1d, b_rel1d, W_root1d,
           W_rel2, b_rel2, W_root2,
           W_rel2d, b_rel2d, W_root2d,
           W_con, b_con, W_out, b_out):
    f32 = jnp.float32
    src_f, dst_f = _prep_edges(edge_index)
    src_r, dst_r = _prep_edges(edge_index_re)
    zblk = jnp.zeros((128, Q), dtype=f32)

    row = lambda b: b.reshape(1, -1)
    segsum = _make_segsum()

    h4, rootf1, rootr1 = _first_tc(x, W_first.T, row(b_first),
                                   W_root1.T, W_root1d.T,
                                   row(b_rel1), row(b_rel1d))
    aggf1, aggr1 = segsum(h4.reshape(4 * NP, Q), src_f, dst_f, src_r, dst_r,
                          zblk)
    h1, rootf2, rootr2 = _layer1_tc(aggf1, aggr1, rootf1, rootr1,
                                    W_rel1.T, W_rel1d.T, W_con.T, row(b_con),
                                    W_root2.T, W_root2d.T,
                                    row(b_rel2), row(b_rel2d))
    aggf2, aggr2 = segsum(h1.reshape(4 * NP, Q), src_f, dst_f, src_r, dst_r,
                          zblk)
    out = _layer2_tc(aggf2, aggr2, rootf2, rootr2,
                     W_rel2.T, W_rel2d.T, W_out.T, row(b_out))
    return out[:N]


# speedup vs baseline: 1.0783x; 1.0783x over previous
import functools

import jax
import jax.numpy as jnp
from jax import lax
from jax.experimental import pallas as pl
from jax.experimental.pallas import tpu as pltpu
from jax.experimental.pallas import tpu_sc as plsc

N = 10000
E = 160000
F = 256
H = 256
C = 16

BM = 256
NP = 10240
NB = NP // BM
EP = 163840
EB = EP // 128
CH = 5
Q = 64
N_SUBCORES = 16
ROWS_PER_TILE = EB // N_SUBCORES
CHUNKS_PER_TILE = ROWS_PER_TILE // CH
STRIPE = NP // N_SUBCORES


def _segsum_body(table, src_f, dst_f, src_r, dst_r, zblk,
                 agg_f, agg_r, idx_s, idx_d, rows, acc,
                 sem_g0, sem_g1, sem_s0, sem_s1):
    c = lax.axis_index("c")
    s = lax.axis_index("s")
    sem_g = (sem_g0, sem_g1)
    sem_s = (sem_s0, sem_s1)

    base = s * ROWS_PER_TILE
    phases = [(src_f, dst_f, agg_f, c * 2),
              (src_r, dst_r, agg_r, c * 2),
              (src_f, dst_f, agg_f, c * 2 + 1),
              (src_r, dst_r, agg_r, c * 2 + 1)]

    def fire_gather(ph, k, b):
        src_ref, dst_ref, _, q = ph
        r0 = base + k * CH
        pltpu.sync_copy(src_ref.at[q, pl.ds(r0, CH)], idx_s.at[b])
        pltpu.sync_copy(dst_ref.at[pl.ds(r0, CH)], idx_d.at[b])
        for j in range(CH):
            pltpu.async_copy(table.at[idx_s.at[b, j]], rows.at[b, j],
                             sem_g[b])

    def drain_gather(b):
        for j in range(CH):
            pltpu.make_async_copy(table.at[idx_s.at[b, j]],
                                  rows.at[b, j], sem_g[b]).wait()

    def fire_scatter(b):
        for j in range(CH):
            pltpu.async_copy(rows.at[b, j], acc.at[idx_d.at[b, j]],
                             sem_s[b], add=True)

    def drain_scatter(b):
        for j in range(CH):
            pltpu.make_async_copy(rows.at[b, j], acc.at[idx_d.at[b, j]],
                                  sem_s[b]).wait()

    def zero_stripe():
        for z in range(STRIPE // 128):
            pltpu.sync_copy(zblk, acc.at[pl.ds(s * STRIPE + z * 128, 128)])

    fire_gather(phases[0], 0, 0)
    zero_stripe()
    plsc.subcore_barrier()

    for pi in range(4):
        ph = phases[pi]
        LAST2 = CHUNKS_PER_TILE - 2

        def outer(i, carry, ph=ph):
            for b in range(2):
                k = 2 * i + b

                @pl.when(k >= 1)
                def _():
                    drain_scatter(1 - b)

                fire_gather(ph, k + 1, 1 - b)
                drain_gather(b)
                fire_scatter(b)
            return carry

        lax.fori_loop(0, LAST2 // 2, outer, 0)
        drain_scatter(1)
        fire_gather(ph, CHUNKS_PER_TILE - 1, 1)
        drain_gather(0)
        fire_scatter(0)
        drain_scatter(0)
        if pi + 1 < 4:
            fire_gather(phases[pi + 1], 0, 0)
        drain_gather(1)
        fire_scatter(1)
        drain_scatter(1)
        plsc.subcore_barrier()
        _, _, out_ref, q = ph
        pltpu.sync_copy(acc.at[pl.ds(s * STRIPE, STRIPE)],
                        out_ref.at[q, pl.ds(s * STRIPE, STRIPE)])
        if pi + 1 < 4:
            zero_stripe()
            plsc.subcore_barrier()


def _make_segsum():
    mesh = plsc.VectorSubcoreMesh(core_axis_name="c", subcore_axis_name="s")
    return pl.kernel(
        _segsum_body,
        out_type=(
            jax.ShapeDtypeStruct((4, NP, Q), jnp.float32),
            jax.ShapeDtypeStruct((4, NP, Q), jnp.float32),
        ),
        mesh=mesh,
        scratch_types=[
            pltpu.VMEM((2, CH, 128), jnp.int32),
            pltpu.VMEM((2, CH, 128), jnp.int32),
            pltpu.VMEM((2, CH, 128, Q), jnp.float32),
            pltpu.VMEM_SHARED((NP, Q), jnp.float32),
            pltpu.SemaphoreType.DMA,
            pltpu.SemaphoreType.DMA,
            pltpu.SemaphoreType.DMA,
            pltpu.SemaphoreType.DMA,
        ],
        compiler_params=pltpu.CompilerParams(use_tc_tiling_on_sc=False),
    )


def _split4(o_ref, h):
    for q in range(4):
        o_ref[q] = h[:, q * Q:(q + 1) * Q]


def _cat(r4):
    return jnp.concatenate([r4[0], r4[1], r4[2], r4[3]], axis=1)


def _first_body(x_ref, w_ref, b_ref, o_ref):
    h = jnp.dot(x_ref[...], w_ref[...], preferred_element_type=jnp.float32)
    h = jnp.maximum(h + b_ref[...], 0.0)
    _split4(o_ref, h)


def _first_tc(x_pad, WfT, bf):
    return pl.pallas_call(
        _first_body,
        grid=(NB,),
        in_specs=[
            pl.BlockSpec((BM, F), lambda i: (i, 0)),
            pl.BlockSpec((F, H), lambda i: (0, 0)),
            pl.BlockSpec((1, H), lambda i: (0, 0)),
        ],
        out_specs=pl.BlockSpec((4, BM, Q), lambda i: (0, i, 0)),
        out_shape=jax.ShapeDtypeStruct((4, NP, Q), jnp.float32),
    )(x_pad, WfT, bf)


def _layer1_body(af_ref, ar_ref, h_ref,
                 wr1_ref, wt1_ref, wr1d_ref, wt1d_ref, wcon_ref,
                 b1_ref, b1d_ref, bcon_ref, o_ref):
    af = _cat(af_ref[...])
    ar = _cat(ar_ref[...])
    h = _cat(h_ref[...])
    dot = functools.partial(jnp.dot, preferred_element_type=jnp.float32)
    x1 = jnp.maximum(dot(af, wr1_ref[...]) + dot(h, wt1_ref[...])
                     + b1_ref[...], 0.0)
    x2 = jnp.maximum(dot(ar, wr1d_ref[...]) + dot(h, wt1d_ref[...])
                     + b1d_ref[...], 0.0)
    wcon = wcon_ref[...]
    h1 = dot(x1, wcon[:H]) + dot(x2, wcon[H:]) + bcon_ref[...]
    _split4(o_ref, h1)


def _layer1_tc(aggf, aggr, h4, Wr1T, Wt1T, Wr1dT, Wt1dT, WconT, b1, b1d, bcon):
    full = lambda a, b: pl.BlockSpec((a, b), lambda i: (0, 0))
    blk4 = pl.BlockSpec((4, BM, Q), lambda i: (0, i, 0))
    return pl.pallas_call(
        _layer1_body,
        grid=(NB,),
        in_specs=[blk4, blk4, blk4,
                  full(H, H), full(H, H), full(H, H), full(H, H),
                  full(2 * H, H), full(1, H), full(1, H), full(1, H)],
        out_specs=pl.BlockSpec((4, BM, Q), lambda i: (0, i, 0)),
        out_shape=jax.ShapeDtypeStruct((4, NP, Q), jnp.float32),
    )(aggf, aggr, h4, Wr1T, Wt1T, Wr1dT, Wt1dT, WconT, b1, b1d, bcon)


def _layer2_body(af_ref, ar_ref, h_ref,
                 wr2_ref, wt2_ref, wr2d_ref, wt2d_ref, wout_ref,
                 b2_ref, b2d_ref, bout_ref, o_ref):
    af = _cat(af_ref[...])
    ar = _cat(ar_ref[...])
    h = _cat(h_ref[...])
    dot = functools.partial(jnp.dot, preferred_element_type=jnp.float32)
    x1 = jnp.maximum(dot(af, wr2_ref[...]) + dot(h, wt2_ref[...])
                     + b2_ref[...], 0.0)
    x2 = jnp.maximum(dot(ar, wr2d_ref[...]) + dot(h, wt2d_ref[...])
                     + b2d_ref[...], 0.0)
    wout = wout_ref[...]
    logits = dot(x1, wout[:H]) + dot(x2, wout[H:]) + bout_ref[...]
    m = jnp.max(logits, axis=-1, keepdims=True)
    z = logits - m
    lse = jnp.log(jnp.sum(jnp.exp(z), axis=-1, keepdims=True))
    o_ref[...] = z - lse


def _layer2_tc(aggf, aggr, h4, Wr2T, Wt2T, Wr2dT, Wt2dT, WoutT, b2, b2d, bout):
    full = lambda a, b: pl.BlockSpec((a, b), lambda i: (0, 0))
    blk4 = pl.BlockSpec((4, BM, Q), lambda i: (0, i, 0))
    return pl.pallas_call(
        _layer2_body,
        grid=(NB,),
        in_specs=[blk4, blk4, blk4,
                  full(H, H), full(H, H), full(H, H), full(H, H),
                  full(2 * H, C), full(1, H), full(1, H), full(1, C)],
        out_specs=pl.BlockSpec((BM, C), lambda i: (i, 0)),
        out_shape=jax.ShapeDtypeStruct((NP, C), jnp.float32),
    )(aggf, aggr, h4, Wr2T, Wt2T, Wr2dT, Wt2dT, WoutT, b2, b2d, bout)


def _prep_edges(ei):
    src = jnp.full((EP,), N, dtype=jnp.int32).at[:E].set(ei[0])
    dst = jnp.full((EP,), N, dtype=jnp.int32).at[:E].set(ei[1])
    offs = jnp.arange(4, dtype=jnp.int32)[:, None] * NP
    srcq = (src[None, :] + offs).reshape(4, EB, 128)
    return srcq, dst.reshape(EB, 128)


def kernel(x, edge_index, edge_weight, edge_index_re, edge_weight_re,
           W_first, b_first,
           W_rel1, b_rel1, W_root1,
           W_rel1d, b_rel1d, W_root1d,
           W_rel2, b_rel2, W_root2,
           W_rel2d, b_rel2d, W_root2d,
           W_con, b_con, W_out, b_out):
    f32 = jnp.float32
    x_pad = jnp.zeros((NP, F), dtype=f32).at[:N].set(x)
    src_f, dst_f = _prep_edges(edge_index)
    src_r, dst_r = _prep_edges(edge_index_re)
    zblk = jnp.zeros((128, Q), dtype=f32)

    row = lambda b: b.reshape(1, -1)
    segsum = _make_segsum()

    h4 = _first_tc(x_pad, W_first.T, row(b_first))
    aggf1, aggr1 = segsum(h4.reshape(4 * NP, Q), src_f, dst_f, src_r, dst_r,
                          zblk)
    h1 = _layer1_tc(aggf1, aggr1, h4,
                    W_rel1.T, W_root1.T, W_rel1d.T, W_root1d.T, W_con.T,
                    row(b_rel1), row(b_rel1d), row(b_con))
    aggf2, aggr2 = segsum(h1.reshape(4 * NP, Q), src_f, dst_f, src_r, dst_r,
                          zblk)
    out = _layer2_tc(aggf2, aggr2, h1,
                     W_rel2.T, W_root2.T, W_rel2d.T, W_root2d.T, W_out.T,
                     row(b_rel2), row(b_rel2d), row(b_out))
    return out[:N]


# speedup vs baseline: 1.1065x; 1.0262x over previous
import functools

import jax
import jax.numpy as jnp
from jax import lax
from jax.experimental import pallas as pl
from jax.experimental.pallas import tpu as pltpu
from jax.experimental.pallas import tpu_sc as plsc

N = 10000
E = 160000
F = 256
H = 256
C = 16

BM = 256
NP = 10240
NB = NP // BM
EP = 163840
EB = EP // 128
CH = 5
Q = 64
N_SUBCORES = 16
ROWS_PER_TILE = EB // N_SUBCORES
CHUNKS_PER_TILE = ROWS_PER_TILE // CH
STRIPE = NP // N_SUBCORES


def _segsum_body(table, src_f, dst_f, src_r, dst_r, zblk,
                 agg_f, agg_r, idx_s, idx_d, rows, acc,
                 sem_g0, sem_g1, sem_s0, sem_s1):
    c = lax.axis_index("c")
    s = lax.axis_index("s")
    sem_g = (sem_g0, sem_g1)
    sem_s = (sem_s0, sem_s1)

    base = s * ROWS_PER_TILE
    phases = [(src_f, dst_f, agg_f, c * 2),
              (src_r, dst_r, agg_r, c * 2),
              (src_f, dst_f, agg_f, c * 2 + 1),
              (src_r, dst_r, agg_r, c * 2 + 1)]

    def fire_gather(ph, k, b):
        src_ref, dst_ref, _, q = ph
        r0 = base + k * CH
        pltpu.sync_copy(src_ref.at[q, pl.ds(r0, CH)], idx_s.at[b])
        pltpu.sync_copy(dst_ref.at[pl.ds(r0, CH)], idx_d.at[b])
        for j in range(CH):
            pltpu.async_copy(table.at[idx_s.at[b, j]], rows.at[b, j],
                             sem_g[b])

    def drain_gather(b):
        for j in range(CH):
            pltpu.make_async_copy(table.at[idx_s.at[b, j]],
                                  rows.at[b, j], sem_g[b]).wait()

    def fire_scatter(b):
        for j in range(CH):
            pltpu.async_copy(rows.at[b, j], acc.at[idx_d.at[b, j]],
                             sem_s[b], add=True)

    def drain_scatter(b):
        for j in range(CH):
            pltpu.make_async_copy(rows.at[b, j], acc.at[idx_d.at[b, j]],
                                  sem_s[b]).wait()

    def zero_stripe():
        pltpu.sync_copy(zblk, acc.at[pl.ds(s * STRIPE, STRIPE)])

    fire_gather(phases[0], 0, 0)
    zero_stripe()
    plsc.subcore_barrier()

    for pi in range(4):
        ph = phases[pi]
        LAST2 = CHUNKS_PER_TILE - 2

        def outer(i, carry, ph=ph):
            for b in range(2):
                k = 2 * i + b

                @pl.when(k >= 1)
                def _():
                    drain_scatter(1 - b)

                fire_gather(ph, k + 1, 1 - b)
                drain_gather(b)
                fire_scatter(b)
            return carry

        lax.fori_loop(0, LAST2 // 2, outer, 0)
        drain_scatter(1)
        fire_gather(ph, CHUNKS_PER_TILE - 1, 1)
        drain_gather(0)
        fire_scatter(0)
        drain_scatter(0)
        if pi + 1 < 4:
            fire_gather(phases[pi + 1], 0, 0)
        drain_gather(1)
        fire_scatter(1)
        drain_scatter(1)
        plsc.subcore_barrier()
        _, _, out_ref, q = ph
        pltpu.sync_copy(acc.at[pl.ds(s * STRIPE, STRIPE)],
                        out_ref.at[q, pl.ds(s * STRIPE, STRIPE)])
        if pi + 1 < 4:
            zero_stripe()
            plsc.subcore_barrier()


def _make_segsum():
    mesh = plsc.VectorSubcoreMesh(core_axis_name="c", subcore_axis_name="s")
    return pl.kernel(
        _segsum_body,
        out_type=(
            jax.ShapeDtypeStruct((4, NP, Q), jnp.float32),
            jax.ShapeDtypeStruct((4, NP, Q), jnp.float32),
        ),
        mesh=mesh,
        scratch_types=[
            pltpu.VMEM((2, CH, 128), jnp.int32),
            pltpu.VMEM((2, CH, 128), jnp.int32),
            pltpu.VMEM((2, CH, 128, Q), jnp.float32),
            pltpu.VMEM_SHARED((NP, Q), jnp.float32),
            pltpu.SemaphoreType.DMA,
            pltpu.SemaphoreType.DMA,
            pltpu.SemaphoreType.DMA,
            pltpu.SemaphoreType.DMA,
        ],
        compiler_params=pltpu.CompilerParams(use_tc_tiling_on_sc=False),
    )


def _split4(o_ref, h):
    for q in range(4):
        o_ref[q] = h[:, q * Q:(q + 1) * Q]


def _cat(r4):
    return jnp.concatenate([r4[0], r4[1], r4[2], r4[3]], axis=1)


def _first_body(x_ref, w_ref, b_ref, o_ref):
    h = jnp.dot(x_ref[...], w_ref[...], preferred_element_type=jnp.float32)
    h = jnp.maximum(h + b_ref[...], 0.0)
    _split4(o_ref, h)


def _first_tc(x_pad, WfT, bf):
    return pl.pallas_call(
        _first_body,
        grid=(NB,),
        in_specs=[
            pl.BlockSpec((BM, F), lambda i: (i, 0)),
            pl.BlockSpec((F, H), lambda i: (0, 0)),
            pl.BlockSpec((1, H), lambda i: (0, 0)),
        ],
        out_specs=pl.BlockSpec((4, BM, Q), lambda i: (0, i, 0)),
        out_shape=jax.ShapeDtypeStruct((4, NP, Q), jnp.float32),
    )(x_pad, WfT, bf)


def _layer1_body(af_ref, ar_ref, h_ref,
                 wr1_ref, wt1_ref, wr1d_ref, wt1d_ref, wcon_ref,
                 b1_ref, b1d_ref, bcon_ref, o_ref):
    af = _cat(af_ref[...])
    ar = _cat(ar_ref[...])
    h = _cat(h_ref[...])
    dot = functools.partial(jnp.dot, preferred_element_type=jnp.float32)
    x1 = jnp.maximum(dot(af, wr1_ref[...]) + dot(h, wt1_ref[...])
                     + b1_ref[...], 0.0)
    x2 = jnp.maximum(dot(ar, wr1d_ref[...]) + dot(h, wt1d_ref[...])
                     + b1d_ref[...], 0.0)
    wcon = wcon_ref[...]
    h1 = dot(x1, wcon[:H]) + dot(x2, wcon[H:]) + bcon_ref[...]
    _split4(o_ref, h1)


def _layer1_tc(aggf, aggr, h4, Wr1T, Wt1T, Wr1dT, Wt1dT, WconT, b1, b1d, bcon):
    full = lambda a, b: pl.BlockSpec((a, b), lambda i: (0, 0))
    blk4 = pl.BlockSpec((4, BM, Q), lambda i: (0, i, 0))
    return pl.pallas_call(
        _layer1_body,
        grid=(NB,),
        in_specs=[blk4, blk4, blk4,
                  full(H, H), full(H, H), full(H, H), full(H, H),
                  full(2 * H, H), full(1, H), full(1, H), full(1, H)],
        out_specs=pl.BlockSpec((4, BM, Q), lambda i: (0, i, 0)),
        out_shape=jax.ShapeDtypeStruct((4, NP, Q), jnp.float32),
    )(aggf, aggr, h4, Wr1T, Wt1T, Wr1dT, Wt1dT, WconT, b1, b1d, bcon)


def _layer2_body(af_ref, ar_ref, h_ref,
                 wr2_ref, wt2_ref, wr2d_ref, wt2d_ref, wout_ref,
                 b2_ref, b2d_ref, bout_ref, o_ref):
    af = _cat(af_ref[...])
    ar = _cat(ar_ref[...])
    h = _cat(h_ref[...])
    dot = functools.partial(jnp.dot, preferred_element_type=jnp.float32)
    x1 = jnp.maximum(dot(af, wr2_ref[...]) + dot(h, wt2_ref[...])
                     + b2_ref[...], 0.0)
    x2 = jnp.maximum(dot(ar, wr2d_ref[...]) + dot(h, wt2d_ref[...])
                     + b2d_ref[...], 0.0)
    wout = wout_ref[...]
    logits = dot(x1, wout[:H]) + dot(x2, wout[H:]) + bout_ref[...]
    m = jnp.max(logits, axis=-1, keepdims=True)
    z = logits - m
    lse = jnp.log(jnp.sum(jnp.exp(z), axis=-1, keepdims=True))
    o_ref[...] = z - lse


def _layer2_tc(aggf, aggr, h4, Wr2T, Wt2T, Wr2dT, Wt2dT, WoutT, b2, b2d, bout):
    full = lambda a, b: pl.BlockSpec((a, b), lambda i: (0, 0))
    blk4 = pl.BlockSpec((4, BM, Q), lambda i: (0, i, 0))
    return pl.pallas_call(
        _layer2_body,
        grid=(NB,),
        in_specs=[blk4, blk4, blk4,
                  full(H, H), full(H, H), full(H, H), full(H, H),
                  full(2 * H, C), full(1, H), full(1, H), full(1, C)],
        out_specs=pl.BlockSpec((BM, C), lambda i: (i, 0)),
        out_shape=jax.ShapeDtypeStruct((NP, C), jnp.float32),
    )(aggf, aggr, h4, Wr2T, Wt2T, Wr2dT, Wt2dT, WoutT, b2, b2d, bout)


def _prep_edges(ei):
    src = jnp.full((EP,), N, dtype=jnp.int32).at[:E].set(ei[0])
    dst = jnp.full((EP,), N, dtype=jnp.int32).at[:E].set(ei[1])
    offs = jnp.arange(4, dtype=jnp.int32)[:, None] * NP
    srcq = (src[None, :] + offs).reshape(4, EB, 128)
    return srcq, dst.reshape(EB, 128)


def kernel(x, edge_index, edge_weight, edge_index_re, edge_weight_re,
           W_first, b_first,
           W_rel1, b_rel1, W_root1,
           W_rel1d, b_rel1d, W_root1d,
           W_rel2, b_rel2, W_root2,
           W_rel2d, b_rel2d, W_root2d,
           W_con, b_con, W_out, b_out):
    f32 = jnp.float32
    x_pad = jnp.zeros((NP, F), dtype=f32).at[:N].set(x)
    src_f, dst_f = _prep_edges(edge_index)
    src_r, dst_r = _prep_edges(edge_index_re)
    zblk = jnp.zeros((STRIPE, Q), dtype=f32)

    row = lambda b: b.reshape(1, -1)
    segsum = _make_segsum()

    h4 = _first_tc(x_pad, W_first.T, row(b_first))
    aggf1, aggr1 = segsum(h4.reshape(4 * NP, Q), src_f, dst_f, src_r, dst_r,
                          zblk)
    h1 = _layer1_tc(aggf1, aggr1, h4,
                    W_rel1.T, W_root1.T, W_rel1d.T, W_root1d.T, W_con.T,
                    row(b_rel1), row(b_rel1d), row(b_con))
    aggf2, aggr2 = segsum(h1.reshape(4 * NP, Q), src_f, dst_f, src_r, dst_r,
                          zblk)
    out = _layer2_tc(aggf2, aggr2, h1,
                     W_rel2.T, W_root2.T, W_rel2d.T, W_root2d.T, W_out.T,
                     row(b_rel2), row(b_rel2d), row(b_out))
    return out[:N]
